# baseline (device time: 46693 ns/iter reference)
import jax
import jax.numpy as jnp
from jax import lax
from jax.experimental import pallas as pl
from jax.experimental.pallas import tpu as pltpu


def kernel(ids, E):
    v_per, d = E.shape
    t_len = ids.shape[0]

    def body(ids_ref, e_ref, out_ref, local_sem, send_sem, recv_sem):
        my_x = lax.axis_index("x")
        my_y = lax.axis_index("y")
        my_z = lax.axis_index("z")
        partner = (my_x, my_y, 1 - my_z)
        row_off = my_z * v_per

        barrier = pltpu.get_barrier_semaphore()
        pl.semaphore_signal(
            barrier, inc=1, device_id=partner,
            device_id_type=pl.DeviceIdType.MESH,
        )
        pl.semaphore_wait(barrier, 1)

        def issue(t, k):
            r = ids_ref[t] - row_off
            own = jnp.logical_and(r >= 0, r < v_per)

            @pl.when(own)
            def _():
                pltpu.make_async_copy(
                    e_ref.at[pl.ds(r, 1), :],
                    out_ref.at[pl.ds(t, 1), :],
                    local_sem,
                ).start()
                pltpu.make_async_remote_copy(
                    src_ref=e_ref.at[pl.ds(r, 1), :],
                    dst_ref=out_ref.at[pl.ds(t, 1), :],
                    send_sem=send_sem,
                    recv_sem=recv_sem,
                    device_id=partner,
                    device_id_type=pl.DeviceIdType.MESH,
                ).start()

            return k + own.astype(jnp.int32)

        n_mine = lax.fori_loop(0, t_len, issue, jnp.int32(0))
        n_peer = t_len - n_mine

        def dummy_rdma():
            return pltpu.make_async_remote_copy(
                src_ref=e_ref.at[pl.ds(0, 1), :],
                dst_ref=out_ref.at[pl.ds(0, 1), :],
                send_sem=send_sem,
                recv_sem=recv_sem,
                device_id=partner,
                device_id_type=pl.DeviceIdType.MESH,
            )

        def wait_recv(i, c):
            dummy_rdma().wait_recv()
            return c

        def wait_send(i, c):
            dummy_rdma().wait_send()
            return c

        def wait_local(i, c):
            pltpu.make_async_copy(
                e_ref.at[pl.ds(0, 1), :],
                out_ref.at[pl.ds(0, 1), :],
                local_sem,
            ).wait()
            return c

        lax.fori_loop(0, n_peer, wait_recv, 0)
        lax.fori_loop(0, n_mine, wait_local, 0)
        lax.fori_loop(0, n_mine, wait_send, 0)

    return pl.pallas_call(
        body,
        out_shape=jax.ShapeDtypeStruct((t_len, d), jnp.float32),
        in_specs=[
            pl.BlockSpec(memory_space=pltpu.SMEM),
            pl.BlockSpec(memory_space=pl.ANY),
        ],
        out_specs=pl.BlockSpec(memory_space=pltpu.VMEM),
        scratch_shapes=[
            pltpu.SemaphoreType.DMA,
            pltpu.SemaphoreType.DMA,
            pltpu.SemaphoreType.DMA,
        ],
        compiler_params=pltpu.CompilerParams(collective_id=11),
    )(ids.astype(jnp.int32), E)


# device time: 41475 ns/iter; 1.1258x vs baseline; 1.1258x over previous
import jax
import jax.numpy as jnp
from jax import lax
from jax.experimental import pallas as pl
from jax.experimental.pallas import tpu as pltpu


def kernel(ids, E):
    v_per, d = E.shape
    t_len = ids.shape[0]
    shift = (v_per - 1).bit_length()

    z = lax.axis_index("z")
    local = ids.astype(jnp.int32) - z * v_per
    mask = (local >= 0) & (local < v_per)
    n_own = jnp.sum(mask.astype(jnp.int32))
    tok = jnp.arange(t_len, dtype=jnp.int32)
    packed = jnp.where(
        mask,
        (tok << shift) | jnp.clip(local, 0, v_per - 1),
        jnp.int32(1 << 30) + tok,
    )
    packed = jnp.sort(packed)
    counts = jnp.stack([n_own, jnp.int32(t_len) - n_own])

    def body(pk_ref, cnt_ref, e_ref, out_ref, local_sem, send_sem, recv_sem):
        my_x = lax.axis_index("x")
        my_y = lax.axis_index("y")
        my_z = lax.axis_index("z")
        partner = (my_x, my_y, 1 - my_z)

        barrier = pltpu.get_barrier_semaphore()
        pl.semaphore_signal(
            barrier, inc=1, device_id=partner,
            device_id_type=pl.DeviceIdType.MESH,
        )
        pl.semaphore_wait(barrier, 1)

        n_mine = cnt_ref[0]
        n_peer = cnt_ref[1]

        def issue(i, c):
            v = pk_ref[i]
            t = v >> shift
            r = v & (v_per - 1)
            pltpu.make_async_copy(
                e_ref.at[pl.ds(r, 1), :],
                out_ref.at[pl.ds(t, 1), :],
                local_sem,
            ).start()
            pltpu.make_async_remote_copy(
                src_ref=e_ref.at[pl.ds(r, 1), :],
                dst_ref=out_ref.at[pl.ds(t, 1), :],
                send_sem=send_sem,
                recv_sem=recv_sem,
                device_id=partner,
                device_id_type=pl.DeviceIdType.MESH,
            ).start()
            return c

        lax.fori_loop(0, n_mine, issue, 0)

        def dummy_rdma():
            return pltpu.make_async_remote_copy(
                src_ref=e_ref.at[pl.ds(0, 1), :],
                dst_ref=out_ref.at[pl.ds(0, 1), :],
                send_sem=send_sem,
                recv_sem=recv_sem,
                device_id=partner,
                device_id_type=pl.DeviceIdType.MESH,
            )

        def wait_recv(i, c):
            dummy_rdma().wait_recv()
            return c

        def wait_send(i, c):
            dummy_rdma().wait_send()
            return c

        def wait_local(i, c):
            pltpu.make_async_copy(
                e_ref.at[pl.ds(0, 1), :],
                out_ref.at[pl.ds(0, 1), :],
                local_sem,
            ).wait()
            return c

        lax.fori_loop(0, n_peer, wait_recv, 0)
        lax.fori_loop(0, n_mine, wait_local, 0)
        lax.fori_loop(0, n_mine, wait_send, 0)

    return pl.pallas_call(
        body,
        out_shape=jax.ShapeDtypeStruct((t_len, d), jnp.float32),
        in_specs=[
            pl.BlockSpec(memory_space=pltpu.SMEM),
            pl.BlockSpec(memory_space=pltpu.SMEM),
            pl.BlockSpec(memory_space=pl.ANY),
        ],
        out_specs=pl.BlockSpec(memory_space=pltpu.VMEM),
        scratch_shapes=[
            pltpu.SemaphoreType.DMA,
            pltpu.SemaphoreType.DMA,
            pltpu.SemaphoreType.DMA,
        ],
        compiler_params=pltpu.CompilerParams(collective_id=11),
    )(packed, counts, E)


# device time: 35575 ns/iter; 1.3125x vs baseline; 1.1658x over previous
import jax
import jax.numpy as jnp
from jax import lax
from jax.experimental import pallas as pl
from jax.experimental.pallas import tpu as pltpu


def kernel(ids, E):
    v_per, d = E.shape
    t_len = ids.shape[0]
    shift = (v_per - 1).bit_length()

    z = lax.axis_index("z")
    local = ids.astype(jnp.int32) - z * v_per
    mask = (local >= 0) & (local < v_per)
    n_own = jnp.sum(mask.astype(jnp.int32))
    tok = jnp.arange(t_len, dtype=jnp.int32)
    packed = jnp.where(
        mask,
        (tok << shift) | jnp.clip(local, 0, v_per - 1),
        jnp.int32(1 << 30) + tok,
    )
    packed = jnp.sort(packed)
    counts = jnp.stack([n_own, jnp.int32(t_len) - n_own])

    def body(pk_ref, cnt_ref, e_ref, out_ref, local_sem, send_sem, recv_sem):
        my_x = lax.axis_index("x")
        my_y = lax.axis_index("y")
        my_z = lax.axis_index("z")
        partner = (my_x, my_y, 1 - my_z)

        barrier = pltpu.get_barrier_semaphore()
        pl.semaphore_signal(
            barrier, inc=1, device_id=partner,
            device_id_type=pl.DeviceIdType.MESH,
        )
        pl.semaphore_wait(barrier, 1)

        n_mine = cnt_ref[0]
        n_peer = cnt_ref[1]

        def issue(i, c):
            v = pk_ref[i]
            t = v >> shift
            r = v & (v_per - 1)
            pltpu.make_async_copy(
                e_ref.at[pl.ds(r, 1), :],
                out_ref.at[pl.ds(t, 1), :],
                local_sem,
            ).start()
            pltpu.make_async_remote_copy(
                src_ref=e_ref.at[pl.ds(r, 1), :],
                dst_ref=out_ref.at[pl.ds(t, 1), :],
                send_sem=send_sem,
                recv_sem=recv_sem,
                device_id=partner,
                device_id_type=pl.DeviceIdType.MESH,
            ).start()
            return c

        lax.fori_loop(0, n_mine, issue, 0)

        def dummy_rdma(k):
            return pltpu.make_async_remote_copy(
                src_ref=e_ref.at[pl.ds(0, k), :],
                dst_ref=out_ref.at[pl.ds(0, k), :],
                send_sem=send_sem,
                recv_sem=recv_sem,
                device_id=partner,
                device_id_type=pl.DeviceIdType.MESH,
            )

        for k in (1024, 512, 256, 128, 64, 32, 16, 8, 4, 2, 1):

            @pl.when((n_peer & k) != 0)
            def _(k=k):
                dummy_rdma(k).wait_recv()

            @pl.when((n_mine & k) != 0)
            def _(k=k):
                pltpu.make_async_copy(
                    e_ref.at[pl.ds(0, k), :],
                    out_ref.at[pl.ds(0, k), :],
                    local_sem,
                ).wait()
                dummy_rdma(k).wait_send()

    return pl.pallas_call(
        body,
        out_shape=jax.ShapeDtypeStruct((t_len, d), jnp.float32),
        in_specs=[
            pl.BlockSpec(memory_space=pltpu.SMEM),
            pl.BlockSpec(memory_space=pltpu.SMEM),
            pl.BlockSpec(memory_space=pl.ANY),
        ],
        out_specs=pl.BlockSpec(memory_space=pltpu.VMEM),
        scratch_shapes=[
            pltpu.SemaphoreType.DMA,
            pltpu.SemaphoreType.DMA,
            pltpu.SemaphoreType.DMA,
        ],
        compiler_params=pltpu.CompilerParams(collective_id=11),
    )(packed, counts, E)


# device time: 34436 ns/iter; 1.3559x vs baseline; 1.0331x over previous
import jax
import jax.numpy as jnp
from jax import lax
from jax.experimental import pallas as pl
from jax.experimental.pallas import tpu as pltpu


def kernel(ids, E):
    v_per, d = E.shape
    t_len = ids.shape[0]
    shift = (v_per - 1).bit_length()

    z = lax.axis_index("z")
    local = ids.astype(jnp.int32) - z * v_per
    mask = (local >= 0) & (local < v_per)
    tok = jnp.arange(t_len, dtype=jnp.int32)
    sentinel = jnp.int32(1 << 30)
    packed = jnp.where(
        mask,
        (tok << shift) | jnp.clip(local, 0, v_per - 1),
        sentinel + tok,
    )
    packed = jnp.sort(packed)

    def body(pk_ref, e_ref, out_ref, local_sem, send_sem, recv_sem):
        my_x = lax.axis_index("x")
        my_y = lax.axis_index("y")
        my_z = lax.axis_index("z")
        partner = (my_x, my_y, 1 - my_z)

        barrier = pltpu.get_barrier_semaphore()
        pl.semaphore_signal(
            barrier, inc=1, device_id=partner,
            device_id_type=pl.DeviceIdType.MESH,
        )
        pl.semaphore_wait(barrier, 1)

        sent = jnp.int32(1 << 30)
        n_mine = jnp.int32(0)
        step = t_len
        while step >= 1:
            cand = n_mine + step
            probe = pk_ref[jnp.minimum(cand, t_len) - 1]
            ok = jnp.logical_and(cand <= t_len, probe < sent)
            n_mine = jnp.where(ok, cand, n_mine)
            step //= 2
        n_peer = t_len - n_mine

        def issue_rdma(i, c):
            v = pk_ref[i]
            pltpu.make_async_remote_copy(
                src_ref=e_ref.at[pl.ds(v & (v_per - 1), 1), :],
                dst_ref=out_ref.at[pl.ds(v >> shift, 1), :],
                send_sem=send_sem,
                recv_sem=recv_sem,
                device_id=partner,
                device_id_type=pl.DeviceIdType.MESH,
            ).start()
            return c

        def issue_local(i, c):
            v = pk_ref[i]
            pltpu.make_async_copy(
                e_ref.at[pl.ds(v & (v_per - 1), 1), :],
                out_ref.at[pl.ds(v >> shift, 1), :],
                local_sem,
            ).start()
            return c

        lax.fori_loop(0, n_mine, issue_rdma, 0)
        lax.fori_loop(0, n_mine, issue_local, 0)

        def dummy_rdma(k):
            return pltpu.make_async_remote_copy(
                src_ref=e_ref.at[pl.ds(0, k), :],
                dst_ref=out_ref.at[pl.ds(0, k), :],
                send_sem=send_sem,
                recv_sem=recv_sem,
                device_id=partner,
                device_id_type=pl.DeviceIdType.MESH,
            )

        for k in (1024, 512, 256, 128, 64, 32, 16, 8, 4, 2, 1):

            @pl.when((n_peer & k) != 0)
            def _(k=k):
                dummy_rdma(k).wait_recv()

            @pl.when((n_mine & k) != 0)
            def _(k=k):
                pltpu.make_async_copy(
                    e_ref.at[pl.ds(0, k), :],
                    out_ref.at[pl.ds(0, k), :],
                    local_sem,
                ).wait()
                dummy_rdma(k).wait_send()

    return pl.pallas_call(
        body,
        out_shape=jax.ShapeDtypeStruct((t_len, d), jnp.float32),
        in_specs=[
            pl.BlockSpec(memory_space=pltpu.SMEM),
            pl.BlockSpec(memory_space=pl.ANY),
        ],
        out_specs=pl.BlockSpec(memory_space=pltpu.VMEM),
        scratch_shapes=[
            pltpu.SemaphoreType.DMA,
            pltpu.SemaphoreType.DMA,
            pltpu.SemaphoreType.DMA,
        ],
        compiler_params=pltpu.CompilerParams(collective_id=11),
    )(packed, E)
